# baseline (speedup 1.0000x reference)
import functools

import jax
import jax.numpy as jnp
from jax.experimental import pallas as pl
from jax.experimental.pallas import tpu as pltpu


def _fused_body(inv_hw, num_classes, C, x_ref, wb_ref, scores_ref, deltas_ref,
                acc_ref):
    k = pl.program_id(1)
    partial = jnp.sum(x_ref[...], axis=0)

    @pl.when(k == 0)
    def _():
        acc_ref[...] = partial

    @pl.when(k != 0)
    def _():
        acc_ref[...] = acc_ref[...] + partial

    @pl.when(k == pl.num_programs(1) - 1)
    def _():
        pooled = acc_ref[...] * inv_hw
        w_all = wb_ref[:, :C]
        b_all = wb_ref[:, C:]
        dn = (((1,), (1,)), ((), ()))
        out = jax.lax.dot_general(
            w_all, pooled, dn,
            preferred_element_type=jnp.float32) + b_all
        scores_ref[...] = out[:num_classes]
        deltas_ref[...] = out[num_classes:]


def kernel(x, w_cls, b_cls, w_pred, b_pred):
    num_classes = w_cls.shape[0]
    nc4 = w_pred.shape[0]
    nout = num_classes + nc4

    if x.ndim == 4:
        N, C, H, W = x.shape
        hw = H * W
        xt = x.transpose(2, 3, 0, 1).reshape(hw, N, C)
    else:
        N, C = x.shape
        hw = 1
        xt = x.reshape(1, N, C)

    w_all = jnp.concatenate([w_cls.astype(jnp.float32),
                             w_pred.astype(jnp.float32)], axis=0)
    b_all = jnp.concatenate([b_cls.astype(jnp.float32),
                             b_pred.astype(jnp.float32)])[:, None]
    wb = jnp.concatenate([w_all, b_all], axis=1)

    if N % 128 == 0:
        tn = 128
    elif N % 8 == 0:
        tn = 8
    else:
        tn = N
    hws = hw
    for cand in (7, 8, 4, 2):
        if hw % cand == 0 and hw // cand > 1:
            hws = hw // cand
            break
    grid = (N // tn, hw // hws)

    itemsize = jnp.dtype(x.dtype).itemsize
    cost = pl.CostEstimate(
        flops=int(N * C * hw + 2 * N * C * nout),
        transcendentals=0,
        bytes_accessed=int(N * C * hw * itemsize + wb.size * 4
                           + N * nout * 4),
    )

    scores_t, deltas_t = pl.pallas_call(
        functools.partial(_fused_body, 1.0 / float(hw), num_classes, C),
        out_shape=(jax.ShapeDtypeStruct((num_classes, N), jnp.float32),
                   jax.ShapeDtypeStruct((nc4, N), jnp.float32)),
        grid=grid,
        in_specs=[
            pl.BlockSpec((hws, tn, C), lambda i, k: (k, i, 0)),
            pl.BlockSpec((nout, C + 1), lambda i, k: (0, 0)),
        ],
        out_specs=[
            pl.BlockSpec((num_classes, tn), lambda i, k: (0, i)),
            pl.BlockSpec((nc4, tn), lambda i, k: (0, i)),
        ],
        scratch_shapes=[pltpu.VMEM((tn, C), jnp.float32)],
        compiler_params=pltpu.CompilerParams(
            dimension_semantics=("parallel", "arbitrary"),
            vmem_limit_bytes=48 * 1024 * 1024,
        ),
        cost_estimate=cost,
    )(xt, wb)
    return scores_t.T, deltas_t.T


# speedup vs baseline: 2.1373x; 2.1373x over previous
import functools

import jax
import jax.numpy as jnp
from jax.experimental import pallas as pl
from jax.experimental.pallas import tpu as pltpu


def _fused_body(inv_hw, num_classes, C, x_ref, wb_ref, scores_ref, deltas_ref):
    pooled = jnp.sum(x_ref[...], axis=0) * inv_hw
    w_all = wb_ref[:, :C]
    b_all = wb_ref[:, C:]
    dn = (((1,), (1,)), ((), ()))
    out = jax.lax.dot_general(
        w_all, pooled, dn,
        preferred_element_type=jnp.float32) + b_all
    scores_ref[...] = out[:num_classes]
    deltas_ref[...] = out[num_classes:]


def kernel(x, w_cls, b_cls, w_pred, b_pred):
    num_classes = w_cls.shape[0]
    nc4 = w_pred.shape[0]
    nout = num_classes + nc4

    if x.ndim == 4:
        N, C, H, W = x.shape
        hw = H * W
        xt = x.transpose(2, 3, 0, 1).reshape(hw, N, C)
    else:
        N, C = x.shape
        hw = 1
        xt = x.reshape(1, N, C)

    w_all = jnp.concatenate([w_cls.astype(jnp.float32),
                             w_pred.astype(jnp.float32)], axis=0)
    b_all = jnp.concatenate([b_cls.astype(jnp.float32),
                             b_pred.astype(jnp.float32)])[:, None]
    wb = jnp.concatenate([w_all, b_all], axis=1)

    if N % 128 == 0:
        tn = 128
    elif N % 8 == 0:
        tn = 8
    else:
        tn = N
    grid = (N // tn,)

    itemsize = jnp.dtype(x.dtype).itemsize
    cost = pl.CostEstimate(
        flops=int(N * C * hw + 2 * N * C * nout),
        transcendentals=0,
        bytes_accessed=int(N * C * hw * itemsize + wb.size * 4
                           + N * nout * 4),
    )

    scores_t, deltas_t = pl.pallas_call(
        functools.partial(_fused_body, 1.0 / float(hw), num_classes, C),
        out_shape=(jax.ShapeDtypeStruct((num_classes, N), jnp.float32),
                   jax.ShapeDtypeStruct((nc4, N), jnp.float32)),
        grid=grid,
        in_specs=[
            pl.BlockSpec((hw, tn, C), lambda i: (0, i, 0)),
            pl.BlockSpec((nout, C + 1), lambda i: (0, 0)),
        ],
        out_specs=[
            pl.BlockSpec((num_classes, tn), lambda i: (0, i)),
            pl.BlockSpec((nc4, tn), lambda i: (0, i)),
        ],
        compiler_params=pltpu.CompilerParams(
            dimension_semantics=("parallel",),
            vmem_limit_bytes=48 * 1024 * 1024,
        ),
        cost_estimate=cost,
    )(xt, wb)
    return scores_t.T, deltas_t.T
